# Initial kernel scaffold; baseline (speedup 1.0000x reference)
import jax
import jax.numpy as jnp
from jax.experimental import pallas as pl


def kernel(log_probs, sequence_scores, inp_tokens):
    raise NotImplementedError("write your pallas kernel here")


# speedup vs baseline: 1.6600x; 1.6600x over previous
import jax
import jax.numpy as jnp
from jax.experimental import pallas as pl

B = 32
BEAM = 16
VOCAB = 100000
EOS = 2
MINUS_INF = -1e20


ROWS = 8
GSUB = 64
GROUPS = 13
SUB = GROUPS * GSUB
PADV = SUB * 128
NEG = -1e30


def _row_kernel(x_ref, seq_ref, tok_ref, vals_ref, toks_ref, y_ref, g_ref):
    y_ref[...] = x_ref[...]
    for g in range(GROUPS):
        g_ref[:, g, :] = jnp.max(x_ref[:, g * GSUB:(g + 1) * GSUB, :], axis=1)

    BIGI = jnp.int32(10**9)
    sub_i = jax.lax.broadcasted_iota(jnp.int32, (GSUB, 128), 0)
    lane_i = jax.lax.broadcasted_iota(jnp.int32, (GSUB, 128), 1)
    p64 = sub_i * 128 + lane_i
    gidx = jax.lax.broadcasted_iota(jnp.int32, (GROUPS, 128), 0)

    col = jax.lax.broadcasted_iota(jnp.int32, (1, BEAM), 1)
    for r in range(ROWS):
        seq_r = seq_ref[r, 0]
        fin_r = tok_ref[r, 0] == EOS
        m_top = jnp.max(g_ref[r])
        log_s = jnp.log(jnp.sum(jnp.exp(x_ref[r] - m_top)))
        vrow = jnp.zeros((1, BEAM), jnp.float32)
        trow = jnp.zeros((1, BEAM), jnp.int32)
        for j in range(BEAM):
            gr = g_ref[r]
            m_r = jnp.max(gr)
            g_r = jnp.min(jnp.where(gr == m_r, gidx, BIGI))
            grp = y_ref[r, pl.ds(g_r * GSUB, GSUB), :]
            p = jnp.min(jnp.where(grp == m_r, p64, BIGI))
            tok_r = g_r * (GSUB * 128) + p
            grp2 = jnp.where(p64 == p, NEG, grp)
            y_ref[r, pl.ds(g_r * GSUB, GSUB), :] = grp2
            g_ref[r, pl.ds(g_r, 1), :] = jnp.max(grp2, axis=0, keepdims=True)
            vrow = jnp.where(col == j, m_r, vrow)
            trow = jnp.where(col == j, tok_r, trow)
        sc_unf = seq_r + ((vrow - m_top) - log_s)
        sc_fin = jnp.where(col == 0, seq_r, MINUS_INF)
        tk_fin = jnp.where(col == 0, EOS, 0)
        vals_ref[r:r + 1, :] = jnp.where(fin_r, sc_fin, sc_unf)
        toks_ref[r:r + 1, :] = jnp.where(fin_r, tk_fin, trow)


def _merge_kernel(s_ref, t_ref, os_ref, op_ref, ot_ref):
    s = s_ref[...]
    t = t_ref[...]
    beam = jax.lax.broadcasted_iota(jnp.int32, (B, BEAM * BEAM), 1) // BEAM
    fkey = beam * VOCAB + t
    big = jnp.int32(2**31 - 1)
    outs, outp, outt = [], [], []
    for _ in range(BEAM):
        m = jnp.max(s, axis=1, keepdims=True)
        k = jnp.min(jnp.where(s == m, fkey, big), axis=1, keepdims=True)
        msk = (s == m) & (fkey == k)
        outs.append(m)
        outp.append(jnp.max(jnp.where(msk, beam, 0), axis=1, keepdims=True))
        outt.append(jnp.max(jnp.where(msk, t, 0), axis=1, keepdims=True))
        s = jnp.where(msk, -jnp.inf, s)
    os_ref[...] = jnp.concatenate(outs, axis=1)
    op_ref[...] = jnp.concatenate(outp, axis=1)
    ot_ref[...] = jnp.concatenate(outt, axis=1)


def kernel(log_probs, sequence_scores, inp_tokens):
    from jax.experimental.pallas import tpu as pltpu

    n = B * BEAM
    pad = PADV - VOCAB
    xp = jnp.concatenate(
        [log_probs, jnp.full((n, pad), NEG, jnp.float32)], axis=1
    ).reshape(n, SUB, 128)
    seq2 = sequence_scores.reshape(n, 1)
    tok2 = inp_tokens.astype(jnp.int32).reshape(n, 1)
    vals, toks = pl.pallas_call(
        _row_kernel,
        grid=(n // ROWS,),
        in_specs=[
            pl.BlockSpec((ROWS, SUB, 128), lambda r: (r, 0, 0)),
            pl.BlockSpec((ROWS, 1), lambda r: (r, 0)),
            pl.BlockSpec((ROWS, 1), lambda r: (r, 0)),
        ],
        out_specs=[
            pl.BlockSpec((ROWS, BEAM), lambda r: (r, 0)),
            pl.BlockSpec((ROWS, BEAM), lambda r: (r, 0)),
        ],
        out_shape=[
            jax.ShapeDtypeStruct((n, BEAM), jnp.float32),
            jax.ShapeDtypeStruct((n, BEAM), jnp.int32),
        ],
        scratch_shapes=[
            pltpu.VMEM((ROWS, SUB, 128), jnp.float32),
            pltpu.VMEM((ROWS, GROUPS, 128), jnp.float32),
        ],
    )(xp, seq2, tok2)

    s = vals.reshape(B, BEAM * BEAM)
    t = toks.reshape(B, BEAM * BEAM)
    scores, preds, tokens = pl.pallas_call(
        _merge_kernel,
        out_shape=[
            jax.ShapeDtypeStruct((B, BEAM), jnp.float32),
            jax.ShapeDtypeStruct((B, BEAM), jnp.int32),
            jax.ShapeDtypeStruct((B, BEAM), jnp.int32),
        ],
    )(s, t)
    return scores, preds, tokens
